# baseline (speedup 1.0000x reference)
import functools

import jax
import jax.numpy as jnp
from jax import lax
from jax.experimental import pallas as pl
from jax.experimental.pallas import tpu as pltpu
from jax.experimental.pallas import tpu_sc as plsc

B = 16384
NF = 32
PACK = 128 // NF
EPS = 1e-5

_NC = 2
_NS = 16
_NW = _NC * _NS
_BPW = B // _NW
_CHUNK = 128
_NCHUNK = _BPW // _CHUNK


_SEG = 8192


def _tc_repack(tt):
    V = tt.shape[1]
    grid = (V + PACK * _SEG - 1) // (PACK * _SEG)

    def body(i0_ref, i1_ref, i2_ref, i3_ref, out_ref):
        for q, r in enumerate((i0_ref, i1_ref, i2_ref, i3_ref)):
            out_ref[:, q * NF:(q + 1) * NF] = r[...].T

    last = (V - 1) // _SEG

    def spec(q):
        return pl.BlockSpec(
            (NF, _SEG), lambda i: (0, jnp.minimum(PACK * i + q, last)))

    return pl.pallas_call(
        body,
        grid=(grid,),
        in_specs=[spec(0), spec(1), spec(2), spec(3)],
        out_specs=pl.BlockSpec((_SEG, 128), lambda i: (i, 0)),
        out_shape=jax.ShapeDtypeStruct((grid * _SEG, 128), jnp.float32),
        compiler_params=pltpu.CompilerParams(
            dimension_semantics=("parallel",),
            fuse_transposed_lhs_in_matmul=True),
    )(tt, tt, tt, tt)


def _sc_gather4(ug_p, eg_p, um_p, em_p, upid2d, epid2d):
    mesh = plsc.VectorSubcoreMesh(core_axis_name="c", subcore_axis_name="s")
    row_t = jax.ShapeDtypeStruct((B, 128), jnp.float32)

    @functools.partial(
        pl.kernel,
        out_type=(row_t, row_t, row_t, row_t),
        mesh=mesh,
        compiler_params=pltpu.CompilerParams(use_tc_tiling_on_sc=False),
        scratch_types=[
            pltpu.VMEM((_NCHUNK, _CHUNK), jnp.int32),
            pltpu.VMEM((_NCHUNK, _CHUNK), jnp.int32),
            pltpu.VMEM((_CHUNK, 128), jnp.float32),
            pltpu.VMEM((_CHUNK, 128), jnp.float32),
            pltpu.VMEM((_CHUNK, 128), jnp.float32),
            pltpu.VMEM((_CHUNK, 128), jnp.float32),
            pltpu.SemaphoreType.DMA,
            pltpu.SemaphoreType.DMA,
            pltpu.SemaphoreType.DMA,
            pltpu.SemaphoreType.DMA,
        ],
    )
    def k(ug_hbm, eg_hbm, um_hbm, em_hbm, uid_hbm, eid_hbm,
          oug_hbm, oeg_hbm, oum_hbm, oem_hbm,
          uidx_v, eidx_v, ug_v, eg_v, um_v, em_v, s1, s2, s3, s4):
        wid = lax.axis_index("s") * _NC + lax.axis_index("c")
        base = wid * _BPW
        pltpu.sync_copy(uid_hbm.at[pl.ds(wid * _NCHUNK, _NCHUNK)], uidx_v)
        pltpu.sync_copy(eid_hbm.at[pl.ds(wid * _NCHUNK, _NCHUNK)], eidx_v)
        for j in range(_NCHUNK):
            cs = pl.ds(base + j * _CHUNK, _CHUNK)
            c1 = pltpu.async_copy(ug_hbm.at[uidx_v.at[j]], ug_v, s1)
            c2 = pltpu.async_copy(eg_hbm.at[eidx_v.at[j]], eg_v, s2)
            c3 = pltpu.async_copy(um_hbm.at[uidx_v.at[j]], um_v, s3)
            c4 = pltpu.async_copy(em_hbm.at[eidx_v.at[j]], em_v, s4)
            c1.wait()
            pltpu.sync_copy(ug_v, oug_hbm.at[cs])
            c2.wait()
            pltpu.sync_copy(eg_v, oeg_hbm.at[cs])
            c3.wait()
            pltpu.sync_copy(um_v, oum_hbm.at[cs])
            c4.wait()
            pltpu.sync_copy(em_v, oem_hbm.at[cs])

    return k(ug_p, eg_p, um_p, em_p, upid2d, epid2d)


def _tc_dense(ug, eg, um, em, phu, phe, w1t, b1, gs1, bt1, w2t, b2, gs2, bt2,
              w3t, b3, gs3, bt3, wp, bp):
    blk = 4096
    grid = B // blk

    def body(ug_ref, eg_ref, um_ref, em_ref, phu_ref, phe_ref,
             w1_ref, b1_ref, gs1_ref, bt1_ref, w2_ref, b2_ref, gs2_ref,
             bt2_ref, w3_ref, b3_ref, gs3_ref, bt3_ref, wp_ref, bp_ref,
             out_ref):
        grp = (jax.lax.broadcasted_iota(jnp.int32, (blk, 128), 1)
               // NF).astype(jnp.float32)
        mu = (grp == phu_ref[...]).astype(jnp.float32)
        me = (grp == phe_ref[...]).astype(jnp.float32)

        def sel(rows, m):
            rm = rows * m
            acc = rm[:, :NF]
            for q in range(1, PACK):
                acc = acc + rm[:, q * NF:(q + 1) * NF]
            return acc

        ug_r = sel(ug_ref[...], mu)
        eg_r = sel(eg_ref[...], me)
        um_r = sel(um_ref[...], mu)
        em_r = sel(em_ref[...], me)
        x = jnp.concatenate([um_r, em_r], axis=1)
        h = jnp.dot(x, w1_ref[...], preferred_element_type=jnp.float32)
        h = jnp.maximum(h + b1_ref[...], 0.0) * gs1_ref[...] + bt1_ref[...]
        h = jnp.dot(h, w2_ref[...], preferred_element_type=jnp.float32)
        h = jnp.maximum(h + b2_ref[...], 0.0) * gs2_ref[...] + bt2_ref[...]
        h = jnp.dot(h, w3_ref[...], preferred_element_type=jnp.float32)
        h = jnp.maximum(h + b3_ref[...], 0.0) * gs3_ref[...] + bt3_ref[...]
        gmf = ug_r * eg_r
        wp_row = wp_ref[...]
        logit = (jnp.sum(gmf * wp_row[:, :NF], axis=1, keepdims=True)
                 + jnp.sum(h * wp_row[:, NF:], axis=1, keepdims=True)
                 + bp_ref[...])
        out_ref[...] = jax.nn.sigmoid(logit)

    def row_spec(shape):
        return pl.BlockSpec((blk,) + shape[1:],
                            lambda i: (i,) + (0,) * (len(shape) - 1))

    def rep_spec(shape):
        return pl.BlockSpec(shape, lambda i: (0,) * len(shape))

    ins = [ug, eg, um, em, phu, phe, w1t, b1, gs1, bt1, w2t, b2, gs2, bt2,
           w3t, b3, gs3, bt3, wp, bp]
    in_specs = [row_spec(a.shape) if a.shape[0] == B else rep_spec(a.shape)
                for a in ins]
    return pl.pallas_call(
        body,
        grid=(grid,),
        in_specs=in_specs,
        out_specs=pl.BlockSpec((blk, 1), lambda i: (i, 0)),
        out_shape=jax.ShapeDtypeStruct((B, 1), jnp.float32),
    )(*ins)


def kernel(user_ids, exercise_ids, ue_gmf, ee_gmf, ue_mlp, ee_mlp,
           W1, b1, g1, bt1, W2, b2, g2, bt2, W3, b3, g3, bt3, Wp, bp):
    uid = user_ids.astype(jnp.int32)
    eid = exercise_ids.astype(jnp.int32)
    seg4 = PACK * _SEG
    upid = _SEG * (uid // seg4) + uid % _SEG
    epid = _SEG * (eid // seg4) + eid % _SEG
    upid2d = upid.reshape(_NW * _NCHUNK, _CHUNK)
    epid2d = epid.reshape(_NW * _NCHUNK, _CHUNK)
    phu = ((uid // _SEG) % PACK).astype(jnp.float32).reshape(B, 1)
    phe = ((eid // _SEG) % PACK).astype(jnp.float32).reshape(B, 1)

    ug_p = _tc_repack(ue_gmf.T)
    eg_p = _tc_repack(ee_gmf.T)
    um_p = _tc_repack(ue_mlp.T)
    em_p = _tc_repack(ee_mlp.T)

    ug, eg, um, em = _sc_gather4(ug_p, eg_p, um_p, em_p, upid2d, epid2d)

    s = 1.0 / jnp.sqrt(jnp.float32(1.0 + EPS))
    args = (ug, eg, um, em, phu, phe,
            W1.T, b1.reshape(1, -1), (g1 * s).reshape(1, -1),
            bt1.reshape(1, -1),
            W2.T, b2.reshape(1, -1), (g2 * s).reshape(1, -1),
            bt2.reshape(1, -1),
            W3.T, b3.reshape(1, -1), (g3 * s).reshape(1, -1),
            bt3.reshape(1, -1),
            Wp, bp.reshape(1, 1))
    out = _tc_dense(*args)
    return out.reshape(B)


# speedup vs baseline: 1.0074x; 1.0074x over previous
import functools

import jax
import jax.numpy as jnp
from jax import lax
from jax.experimental import pallas as pl
from jax.experimental.pallas import tpu as pltpu
from jax.experimental.pallas import tpu_sc as plsc

B = 16384
NF = 32
PACK = 128 // NF
EPS = 1e-5

_NC = 2
_NS = 16
_NW = _NC * _NS
_BPW = B // _NW
_CHUNK = 128
_NCHUNK = _BPW // _CHUNK


_SEG = 4096


def _tc_repack(tt):
    V = tt.shape[1]
    grid = (V + PACK * _SEG - 1) // (PACK * _SEG)

    def body(i0_ref, i1_ref, i2_ref, i3_ref, out_ref):
        for q, r in enumerate((i0_ref, i1_ref, i2_ref, i3_ref)):
            out_ref[:, q * NF:(q + 1) * NF] = r[...].T

    last = (V - 1) // _SEG

    def spec(q):
        return pl.BlockSpec(
            (NF, _SEG), lambda i: (0, jnp.minimum(PACK * i + q, last)))

    return pl.pallas_call(
        body,
        grid=(grid,),
        in_specs=[spec(0), spec(1), spec(2), spec(3)],
        out_specs=pl.BlockSpec((_SEG, 128), lambda i: (i, 0)),
        out_shape=jax.ShapeDtypeStruct((grid * _SEG, 128), jnp.float32),
        compiler_params=pltpu.CompilerParams(
            dimension_semantics=("parallel",),
            fuse_transposed_lhs_in_matmul=True),
    )(tt, tt, tt, tt)


def _sc_gather4(ug_p, eg_p, um_p, em_p, upid2d, epid2d):
    mesh = plsc.VectorSubcoreMesh(core_axis_name="c", subcore_axis_name="s")
    row_t = jax.ShapeDtypeStruct((B, 128), jnp.float32)

    @functools.partial(
        pl.kernel,
        out_type=(row_t, row_t, row_t, row_t),
        mesh=mesh,
        compiler_params=pltpu.CompilerParams(use_tc_tiling_on_sc=False),
        scratch_types=[
            pltpu.VMEM((_NCHUNK, _CHUNK), jnp.int32),
            pltpu.VMEM((_NCHUNK, _CHUNK), jnp.int32),
            pltpu.VMEM((_CHUNK, 128), jnp.float32),
            pltpu.VMEM((_CHUNK, 128), jnp.float32),
            pltpu.VMEM((_CHUNK, 128), jnp.float32),
            pltpu.VMEM((_CHUNK, 128), jnp.float32),
            pltpu.SemaphoreType.DMA,
            pltpu.SemaphoreType.DMA,
            pltpu.SemaphoreType.DMA,
            pltpu.SemaphoreType.DMA,
        ],
    )
    def k(ug_hbm, eg_hbm, um_hbm, em_hbm, uid_hbm, eid_hbm,
          oug_hbm, oeg_hbm, oum_hbm, oem_hbm,
          uidx_v, eidx_v, ug_v, eg_v, um_v, em_v, s1, s2, s3, s4):
        wid = lax.axis_index("s") * _NC + lax.axis_index("c")
        base = wid * _BPW
        pltpu.sync_copy(uid_hbm.at[pl.ds(wid * _NCHUNK, _NCHUNK)], uidx_v)
        pltpu.sync_copy(eid_hbm.at[pl.ds(wid * _NCHUNK, _NCHUNK)], eidx_v)
        for j in range(_NCHUNK):
            cs = pl.ds(base + j * _CHUNK, _CHUNK)
            c1 = pltpu.async_copy(ug_hbm.at[uidx_v.at[j]], ug_v, s1)
            c2 = pltpu.async_copy(eg_hbm.at[eidx_v.at[j]], eg_v, s2)
            c3 = pltpu.async_copy(um_hbm.at[uidx_v.at[j]], um_v, s3)
            c4 = pltpu.async_copy(em_hbm.at[eidx_v.at[j]], em_v, s4)
            c1.wait()
            pltpu.sync_copy(ug_v, oug_hbm.at[cs])
            c2.wait()
            pltpu.sync_copy(eg_v, oeg_hbm.at[cs])
            c3.wait()
            pltpu.sync_copy(um_v, oum_hbm.at[cs])
            c4.wait()
            pltpu.sync_copy(em_v, oem_hbm.at[cs])

    return k(ug_p, eg_p, um_p, em_p, upid2d, epid2d)


def _tc_dense(ug, eg, um, em, phu, phe, w1t, b1, gs1, bt1, w2t, b2, gs2, bt2,
              w3t, b3, gs3, bt3, wp, bp):
    blk = 2048
    grid = B // blk

    def body(ug_ref, eg_ref, um_ref, em_ref, phu_ref, phe_ref,
             w1_ref, b1_ref, gs1_ref, bt1_ref, w2_ref, b2_ref, gs2_ref,
             bt2_ref, w3_ref, b3_ref, gs3_ref, bt3_ref, wp_ref, bp_ref,
             out_ref):
        grp = (jax.lax.broadcasted_iota(jnp.int32, (blk, 128), 1)
               // NF).astype(jnp.float32)
        mu = (grp == phu_ref[...]).astype(jnp.float32)
        me = (grp == phe_ref[...]).astype(jnp.float32)

        def sel(rows, m):
            rm = rows * m
            acc = rm[:, :NF]
            for q in range(1, PACK):
                acc = acc + rm[:, q * NF:(q + 1) * NF]
            return acc

        ug_r = sel(ug_ref[...], mu)
        eg_r = sel(eg_ref[...], me)
        um_r = sel(um_ref[...], mu)
        em_r = sel(em_ref[...], me)
        x = jnp.concatenate([um_r, em_r], axis=1)
        h = jnp.dot(x, w1_ref[...], preferred_element_type=jnp.float32)
        h = jnp.maximum(h + b1_ref[...], 0.0) * gs1_ref[...] + bt1_ref[...]
        h = jnp.dot(h, w2_ref[...], preferred_element_type=jnp.float32)
        h = jnp.maximum(h + b2_ref[...], 0.0) * gs2_ref[...] + bt2_ref[...]
        h = jnp.dot(h, w3_ref[...], preferred_element_type=jnp.float32)
        h = jnp.maximum(h + b3_ref[...], 0.0) * gs3_ref[...] + bt3_ref[...]
        gmf = ug_r * eg_r
        wp_row = wp_ref[...]
        logit = (jnp.sum(gmf * wp_row[:, :NF], axis=1, keepdims=True)
                 + jnp.sum(h * wp_row[:, NF:], axis=1, keepdims=True)
                 + bp_ref[...])
        out_ref[...] = jax.nn.sigmoid(logit)

    def row_spec(shape):
        return pl.BlockSpec((blk,) + shape[1:],
                            lambda i: (i,) + (0,) * (len(shape) - 1))

    def rep_spec(shape):
        return pl.BlockSpec(shape, lambda i: (0,) * len(shape))

    ins = [ug, eg, um, em, phu, phe, w1t, b1, gs1, bt1, w2t, b2, gs2, bt2,
           w3t, b3, gs3, bt3, wp, bp]
    in_specs = [row_spec(a.shape) if a.shape[0] == B else rep_spec(a.shape)
                for a in ins]
    return pl.pallas_call(
        body,
        grid=(grid,),
        in_specs=in_specs,
        out_specs=pl.BlockSpec((blk, 1), lambda i: (i, 0)),
        out_shape=jax.ShapeDtypeStruct((B, 1), jnp.float32),
    )(*ins)


def kernel(user_ids, exercise_ids, ue_gmf, ee_gmf, ue_mlp, ee_mlp,
           W1, b1, g1, bt1, W2, b2, g2, bt2, W3, b3, g3, bt3, Wp, bp):
    uid = user_ids.astype(jnp.int32)
    eid = exercise_ids.astype(jnp.int32)
    seg4 = PACK * _SEG
    upid = _SEG * (uid // seg4) + uid % _SEG
    epid = _SEG * (eid // seg4) + eid % _SEG
    upid2d = upid.reshape(_NW * _NCHUNK, _CHUNK)
    epid2d = epid.reshape(_NW * _NCHUNK, _CHUNK)
    phu = ((uid // _SEG) % PACK).astype(jnp.float32).reshape(B, 1)
    phe = ((eid // _SEG) % PACK).astype(jnp.float32).reshape(B, 1)

    ug_p = _tc_repack(ue_gmf.T)
    eg_p = _tc_repack(ee_gmf.T)
    um_p = _tc_repack(ue_mlp.T)
    em_p = _tc_repack(ee_mlp.T)

    ug, eg, um, em = _sc_gather4(ug_p, eg_p, um_p, em_p, upid2d, epid2d)

    s = 1.0 / jnp.sqrt(jnp.float32(1.0 + EPS))
    args = (ug, eg, um, em, phu, phe,
            W1.T, b1.reshape(1, -1), (g1 * s).reshape(1, -1),
            bt1.reshape(1, -1),
            W2.T, b2.reshape(1, -1), (g2 * s).reshape(1, -1),
            bt2.reshape(1, -1),
            W3.T, b3.reshape(1, -1), (g3 * s).reshape(1, -1),
            bt3.reshape(1, -1),
            Wp, bp.reshape(1, 1))
    out = _tc_dense(*args)
    return out.reshape(B)
